# baseline (speedup 1.0000x reference)
import jax
import jax.numpy as jnp
from jax import lax
from jax.experimental import pallas as pl
from jax.experimental.pallas import tpu as pltpu
from jax.experimental.pallas import tpu_sc as plsc

_N_CLASSES = 100000
_D = 64
_DP = 128
_BATCH = 1024
_INV_TEMP = 20.0
_LOG2E = 1.4426950408889634
_EXP2_SCALE = _INV_TEMP * _LOG2E
_BLK = 4000
_NSTEPS = _N_CLASSES // _BLK

_NC = 2
_NS = 16
_NW = _NC * _NS
_B_PER_W = _BATCH // _NW
_LANES = 16


def _idiv(t16, n):
    return ((t16.astype(jnp.float32) + 0.5) * (1.0 / n)).astype(jnp.int32)


def _sc_gather_body(table_hbm, idx_hbm, out_hbm, idx_v, grp_v, rows_v, out_v, sem):
    q = _BLK // 4
    wid = lax.axis_index("s") * _NC + lax.axis_index("c")
    base = wid * _B_PER_W
    pltpu.sync_copy(idx_hbm.at[pl.ds(base, _B_PER_W)], idx_v)
    for c in range(_B_PER_W // _LANES):
        t16 = idx_v[pl.ds(c * _LANES, _LANES)]
        b16 = _idiv(t16, _BLK)
        r16 = t16 - b16 * _BLK
        q16 = _idiv(r16, q)
        pos16 = r16 - q16 * q
        grp_v[pl.ds(c * _LANES, _LANES)] = b16 * q + pos16
    pltpu.async_copy(table_hbm.at[grp_v], rows_v, sem).wait()
    lane = lax.iota(jnp.int32, _LANES)
    for c in range(_B_PER_W // _LANES):
        t16 = idx_v[pl.ds(c * _LANES, _LANES)]
        b16 = _idiv(t16, _BLK)
        q16 = _idiv(t16 - b16 * _BLK, q)
        h16 = jnp.where(q16 >= 2, _D, 0)
        lo16 = jnp.bitwise_and(q16, 1)
        i16 = lane + (c * _LANES)

        for j in range(_D):
            j16 = jnp.full((_LANES,), j, jnp.int32)
            w = plsc.load_gather(rows_v, [i16, h16 + j16])
            shifted = jnp.where(lo16 == 1, lax.shift_left(w, jnp.int32(16)), w)
            v = jnp.bitwise_and(shifted, jnp.int32(-65536))
            plsc.store_scatter(out_v, [i16, j16], plsc.bitcast(v, jnp.float32))
    pltpu.sync_copy(out_v, out_hbm.at[pl.ds(base, _B_PER_W)])


def _sc_gather(fpairs, targets):
    run = pl.kernel(
        _sc_gather_body,
        out_type=jax.ShapeDtypeStruct((_BATCH, _D), jnp.float32),
        mesh=plsc.VectorSubcoreMesh(
            core_axis_name="c", subcore_axis_name="s",
            num_cores=_NC, num_subcores=_NS),
        scratch_types=[
            pltpu.VMEM((_B_PER_W,), jnp.int32),
            pltpu.VMEM((_B_PER_W,), jnp.int32),
            pltpu.VMEM((_B_PER_W, _DP), jnp.int32),
            pltpu.VMEM((_B_PER_W, _D), jnp.float32),
            pltpu.SemaphoreType.DMA,
        ],
        compiler_params=pltpu.CompilerParams(needs_layout_passes=False),
    )
    return run(fpairs, targets)


def _tc_sumexp_body(xT_ref, f_ref, acc_ref, rinv_ref, fpr_ref, xnTb_ref):
    i = pl.program_id(0)

    @pl.when(i == 0)
    def _init():
        xT = xT_ref[...]
        n2 = jnp.sum(xT * xT, axis=0, keepdims=True)
        r = lax.rsqrt(jnp.maximum(n2, 1e-24))
        rinv_ref[...] = r
        xnTb_ref[...] = (xT * (r * _EXP2_SCALE)).astype(jnp.bfloat16)
        acc_ref[...] = jnp.zeros_like(acc_ref)

    fb = f_ref[...]
    fb16 = fb.astype(jnp.bfloat16)
    qq = _BLK // 4

    def _pack(hi, lo):
        hw = lax.bitcast_convert_type(hi, jnp.uint16).astype(jnp.uint32)
        lw = lax.bitcast_convert_type(lo, jnp.uint16).astype(jnp.uint32)
        return lax.bitcast_convert_type(
            jnp.bitwise_or(lax.shift_left(hw, jnp.uint32(16)), lw), jnp.int32)

    fpr_ref[:, 0:_D] = _pack(fb16[0:qq], fb16[qq:2 * qq])
    fpr_ref[:, _D:_DP] = _pack(fb16[2 * qq:3 * qq], fb16[3 * qq:4 * qq])
    logits = lax.dot_general(
        fb16, xnTb_ref[...],
        (((1,), (0,)), ((), ())),
        preferred_element_type=jnp.float32,
    )
    e = jnp.exp2(logits)
    acc_ref[...] += jnp.sum(e, axis=0, keepdims=True)


def _tc_sumexp(xT, features):
    return pl.pallas_call(
        _tc_sumexp_body,
        grid=(_NSTEPS,),
        in_specs=[
            pl.BlockSpec((_D, _BATCH), lambda i: (0, 0)),
            pl.BlockSpec((_BLK, _D), lambda i: (i, 0)),
        ],
        out_specs=[
            pl.BlockSpec((1, _BATCH), lambda i: (0, 0)),
            pl.BlockSpec((1, _BATCH), lambda i: (0, 0)),
            pl.BlockSpec((_BLK // 4, _DP), lambda i: (i, 0)),
        ],
        out_shape=[
            jax.ShapeDtypeStruct((1, _BATCH), jnp.float32),
            jax.ShapeDtypeStruct((1, _BATCH), jnp.float32),
            jax.ShapeDtypeStruct((_N_CLASSES // 4, _DP), jnp.int32),
        ],
        scratch_shapes=[pltpu.VMEM((_D, _BATCH), jnp.bfloat16)],
        compiler_params=pltpu.CompilerParams(
            dimension_semantics=("arbitrary",),
        ),
    )(xT, features)


def _tc_combine_body(acc_ref, rinv_ref, xTp_ref, gT_ref, out_ref):
    tl_sum = jnp.sum(
        xTp_ref[...] * gT_ref[...] * rinv_ref[...],
        axis=(0, 1), keepdims=True)
    lse_sum = jnp.sum(jnp.log(acc_ref[...]), axis=(0, 1), keepdims=True)
    out_ref[...] = (lse_sum - tl_sum * _INV_TEMP) * (1.0 / _BATCH)


def _tc_combine(acc, rinv, xTp, gT):
    return pl.pallas_call(
        _tc_combine_body,
        out_shape=jax.ShapeDtypeStruct((1, 1), jnp.float32),
    )(acc, rinv, xTp, gT)


def kernel(inputs, features, targets, uncertain_num):
    del uncertain_num
    xT = jnp.transpose(inputs)
    acc, rinv, fpairs = _tc_sumexp(xT, features)
    g = _sc_gather(fpairs, targets)
    gT = jnp.transpose(g)
    loss = _tc_combine(acc, rinv, xT, gT)[0, 0]
    zero = jnp.zeros((1,), jnp.float32)
    return (loss, zero, zero)


# speedup vs baseline: 1.0168x; 1.0168x over previous
import jax
import jax.numpy as jnp
from jax import lax
from jax.experimental import pallas as pl
from jax.experimental.pallas import tpu as pltpu
from jax.experimental.pallas import tpu_sc as plsc

_N_CLASSES = 100000
_D = 64
_DP = 128
_BATCH = 1024
_INV_TEMP = 20.0
_LOG2E = 1.4426950408889634
_EXP2_SCALE = _INV_TEMP * _LOG2E
_BLK = 4000
_NSTEPS = _N_CLASSES // _BLK

_NC = 2
_NS = 16
_NW = _NC * _NS
_B_PER_W = _BATCH // _NW
_LANES = 16


def _idiv_blk(t16):
    return ((t16.astype(jnp.float32) + 0.5) * (1.0 / _BLK)).astype(jnp.int32)


def _sc_gather_body(table_hbm, idx_hbm, out_hbm, idx_v, grp_v, rows_v, out_v, sem):
    wid = lax.axis_index("s") * _NC + lax.axis_index("c")
    base = wid * _B_PER_W
    pltpu.sync_copy(idx_hbm.at[pl.ds(base, _B_PER_W)], idx_v)
    for c in range(_B_PER_W // _LANES):
        t16 = idx_v[pl.ds(c * _LANES, _LANES)]
        b16 = _idiv_blk(t16)
        r16 = t16 - b16 * _BLK
        pos16 = jnp.where(r16 >= _BLK // 2, r16 - _BLK // 2, r16)
        grp_v[pl.ds(c * _LANES, _LANES)] = b16 * (_BLK // 2) + pos16
    pltpu.async_copy(table_hbm.at[grp_v], rows_v, sem).wait()
    lane = lax.iota(jnp.int32, _LANES)
    for c in range(_B_PER_W // _LANES):
        t16 = idx_v[pl.ds(c * _LANES, _LANES)]
        r16 = t16 - _idiv_blk(t16) * _BLK
        h16 = jnp.where(r16 >= _BLK // 2, _D, 0)
        i16 = lane * 0 + jnp.arange(_LANES, dtype=jnp.int32) + (c * _LANES)

        def sel(j, _):
            j16 = jnp.full((_LANES,), 0, jnp.int32) + j
            vals = plsc.load_gather(rows_v, [i16, h16 + j16])
            plsc.store_scatter(out_v, [i16, j16], vals)
            return 0

        lax.fori_loop(0, _D, sel, 0)
    pltpu.sync_copy(out_v, out_hbm.at[pl.ds(base, _B_PER_W)])


def _sc_gather(fpairs, targets):
    run = pl.kernel(
        _sc_gather_body,
        out_type=jax.ShapeDtypeStruct((_BATCH, _D), jnp.float32),
        mesh=plsc.VectorSubcoreMesh(
            core_axis_name="c", subcore_axis_name="s",
            num_cores=_NC, num_subcores=_NS),
        scratch_types=[
            pltpu.VMEM((_B_PER_W,), jnp.int32),
            pltpu.VMEM((_B_PER_W,), jnp.int32),
            pltpu.VMEM((_B_PER_W, _DP), jnp.float32),
            pltpu.VMEM((_B_PER_W, _D), jnp.float32),
            pltpu.SemaphoreType.DMA,
        ],
        compiler_params=pltpu.CompilerParams(needs_layout_passes=False),
    )
    return run(fpairs, targets)


def _tc_sumexp_body(xT_ref, f_ref, acc_ref, rinv_ref, fpr_ref, xnTb_ref):
    i = pl.program_id(0)

    @pl.when(i == 0)
    def _init():
        xT = xT_ref[...]
        n2 = jnp.sum(xT * xT, axis=0, keepdims=True)
        r = lax.rsqrt(jnp.maximum(n2, 1e-24))
        rinv_ref[...] = r
        xnTb_ref[...] = (xT * (r * _EXP2_SCALE)).astype(jnp.bfloat16)
        acc_ref[...] = jnp.zeros_like(acc_ref)

    fb = f_ref[...]
    fpr_ref[:, 0:_D] = fb[0:_BLK // 2, :]
    fpr_ref[:, _D:_DP] = fb[_BLK // 2:_BLK, :]
    fb16 = fb.astype(jnp.bfloat16)
    logits = lax.dot_general(
        fb16, xnTb_ref[...],
        (((1,), (0,)), ((), ())),
        preferred_element_type=jnp.float32,
    )
    e = jnp.exp2(logits)
    acc_ref[...] += jnp.sum(e, axis=0, keepdims=True)


def _tc_sumexp(xT, features):
    return pl.pallas_call(
        _tc_sumexp_body,
        grid=(_NSTEPS,),
        in_specs=[
            pl.BlockSpec((_D, _BATCH), lambda i: (0, 0)),
            pl.BlockSpec((_BLK, _D), lambda i: (i, 0)),
        ],
        out_specs=[
            pl.BlockSpec((1, _BATCH), lambda i: (0, 0)),
            pl.BlockSpec((1, _BATCH), lambda i: (0, 0)),
            pl.BlockSpec((_BLK // 2, _DP), lambda i: (i, 0)),
        ],
        out_shape=[
            jax.ShapeDtypeStruct((1, _BATCH), jnp.float32),
            jax.ShapeDtypeStruct((1, _BATCH), jnp.float32),
            jax.ShapeDtypeStruct((_N_CLASSES // 2, _DP), jnp.float32),
        ],
        scratch_shapes=[pltpu.VMEM((_D, _BATCH), jnp.bfloat16)],
        compiler_params=pltpu.CompilerParams(
            dimension_semantics=("arbitrary",),
        ),
    )(xT, features)


def _tc_combine_body(acc_ref, rinv_ref, xTp_ref, gT_ref, out_ref):
    tl_sum = jnp.sum(
        xTp_ref[...] * gT_ref[...] * rinv_ref[...],
        axis=(0, 1), keepdims=True)
    lse_sum = jnp.sum(jnp.log(acc_ref[...]), axis=(0, 1), keepdims=True)
    out_ref[...] = (lse_sum - tl_sum * _INV_TEMP) * (1.0 / _BATCH)


def _tc_combine(acc, rinv, xTp, gT):
    return pl.pallas_call(
        _tc_combine_body,
        out_shape=jax.ShapeDtypeStruct((1, 1), jnp.float32),
    )(acc, rinv, xTp, gT)


def kernel(inputs, features, targets, uncertain_num):
    del uncertain_num
    xT = jnp.transpose(inputs)
    acc, rinv, fpairs = _tc_sumexp(xT, features)
    g = _sc_gather(fpairs, targets)
    gT = jnp.transpose(g)
    loss = _tc_combine(acc, rinv, xT, gT)[0, 0]
    zero = jnp.zeros((1,), jnp.float32)
    return (loss, zero, zero)
